# baseline (speedup 1.0000x reference)
import functools

import jax
import jax.numpy as jnp
from jax import lax
from jax.experimental import pallas as pl
from jax.experimental.pallas import tpu as pltpu
from jax.experimental.pallas import tpu_sc as plsc

NROWS = 128
NCOLS = 32768
K_SEL = 200
L = 16
NC = 2
NS = 16
NW = NC * NS
ROWS_PER_W = NROWS // NW
NV = NCOLS // L
UNROLL = 32
NOUTER = NV // UNROLL

NB_A = 512
NB_S = 128
LOW_MASK = (1 << 21) - 1


def _scan_hist(h_ref, nbuckets, rank, iota, fifteen):
    def body(j, carry):
        cum, found, bucket, cb = carry
        h = h_ref[pl.ds(j * L, L)]
        pc = plsc.cumsum(h)
        tot = pc + cum
        crossed = tot >= rank
        has = plsc.all_reduce_population_count(crossed) > 0
        ffs = plsc.all_reduce_ffs(crossed)
        is_new = jnp.logical_and(has, jnp.logical_not(found))
        bucket = jnp.where(is_new, j * L + ffs, bucket)
        cb = jnp.where(jnp.logical_and(is_new, iota == ffs), tot - h, cb)
        found = jnp.logical_or(found, has)
        cum = cum + jnp.take(pc, fifteen, mode="wrap")
        return cum, found, bucket, cb

    zs = jnp.zeros((L,), jnp.int32)
    init = (zs, zs > 0, zs, zs)
    _, _, bucket, cb = lax.fori_loop(0, nbuckets // L, body, init)
    return bucket, cb


def _body(
    scores_hbm,
    delta_hbm,
    out_hbm,
    rv0,
    rv1,
    buf,
    ha0,
    ha1,
    ha2,
    ha3,
    hs,
    delta_v,
    si0,
    si1,
    so0,
    so1,
):
    wid = lax.axis_index("s") * NC + lax.axis_index("c")
    pltpu.sync_copy(delta_hbm, delta_v)
    ones_i = jnp.ones((L,), jnp.int32)
    zeros_i = jnp.zeros((L,), jnp.int32)
    iota = lax.iota(jnp.int32, L)
    fifteen = jnp.full((L,), L - 1, jnp.int32)
    rank0 = jnp.full((L,), K_SEL, jnp.int32)
    big_i = jnp.full((L,), 0x7FFFFFFF, jnp.int32)

    def refine_level(shift, path, rank, nvb):
        for z in range(NB_S // L):
            hs[pl.ds(z * L, L)] = zeros_i

        def hist(j, _):
            low = buf[pl.ds(j * L, L)]
            m = lax.shift_right_logical(low, shift + 7) == path
            key = jnp.bitwise_and(lax.shift_right_logical(low, shift), NB_S - 1)
            plsc.addupdate_scatter(hs, [key], ones_i, mask=m)
            return 0

        lax.fori_loop(0, nvb, hist, 0)
        b, cb = _scan_hist(hs, NB_S, rank, iota, fifteen)
        return path * NB_S + b, rank - jnp.sum(cb)

    def process_row(row_v):
        has = (ha0, ha1, ha2, ha3)

        def zero_a(z, _):
            for h in has:
                h[pl.ds(z * L, L)] = zeros_i
            return 0

        lax.fori_loop(0, NB_A // L, zero_a, 0)

        def pass_a(i, _):
            bs = [
                lax.shift_right_logical(
                    lax.bitcast_convert_type(
                        row_v[pl.ds(i * (L * UNROLL) + u * L, L)], jnp.int32
                    ),
                    21,
                )
                for u in range(UNROLL)
            ]
            for u in range(UNROLL):
                plsc.addupdate_scatter(has[u % 4], [bs[u]], ones_i)
            return 0

        lax.fori_loop(0, NOUTER, pass_a, 0)

        def merge(j, _):
            sl = pl.ds(j * L, L)
            ha0[sl] = (ha0[sl] + ha1[sl]) + (ha2[sl] + ha3[sl])
            return 0

        lax.fori_loop(0, NB_A // L, merge, 0)
        pa, cb_a = _scan_hist(ha0, NB_A, rank0, iota, fifteen)
        rank = rank0 - jnp.sum(cb_a)

        def pass_b(i, off):
            bits_l = [
                lax.bitcast_convert_type(
                    row_v[pl.ds(i * (L * UNROLL) + u * L, L)], jnp.int32
                )
                for u in range(UNROLL)
            ]
            match_l = [lax.shift_right_logical(b, 21) == pa for b in bits_l]
            adv_l = [
                jnp.where(plsc.all_reduce_population_count(m) > 0, L, 0)
                for m in match_l
            ]
            offs = [off]
            for u in range(1, UNROLL):
                offs.append(offs[-1] + adv_l[u - 1])
            for u in range(UNROLL):
                plsc.store_scatter(buf, [offs[u] + iota], bits_l[u])
            return offs[-1] + adv_l[-1]

        off = lax.fori_loop(0, NOUTER, pass_b, zeros_i)
        nvb = jnp.max(off) // L

        path = pa
        path, rank = refine_level(14, path, rank, nvb)
        path, rank = refine_level(7, path, rank, nvb)
        path, _ = refine_level(0, path, rank, nvb)

        bits200 = path
        x200 = lax.bitcast_convert_type(bits200, jnp.float32)
        y200 = 1.0 - (2.0 * x200) / 0.05
        kth = (jnp.where(y200 > 0.0, y200, jnp.exp(y200) - 1.0) + 1.0) / 2.0
        kth = kth + delta_v[...]
        kth_s = jnp.max(kth)

        @pl.when(kth_s >= 0.5)
        def _():
            def out_fast(i, _):
                for u in range(UNROLL):
                    sl = pl.ds(i * (L * UNROLL) + u * L, L)
                    y = 1.0 - (2.0 * row_v[sl]) / 0.05
                    f = (y + 1.0) / 2.0
                    row_v[sl] = jnp.where(f >= kth, f, 0.0)
                return 0

            lax.fori_loop(0, NOUTER, out_fast, 0)

        @pl.when(kth_s < 0.5)
        def _():
            def out_full(i, _):
                for u in range(UNROLL):
                    sl = pl.ds(i * (L * UNROLL) + u * L, L)
                    y = 1.0 - (2.0 * row_v[sl]) / 0.05
                    c = jnp.where(y > 0.0, y, jnp.exp(y) - 1.0)
                    f = (c + 1.0) / 2.0
                    row_v[sl] = jnp.where(f >= kth, f, 0.0)
                return 0

            lax.fori_loop(0, NOUTER, out_full, 0)

    rbufs = (rv0, rv1)
    in_sems = (si0, si1)
    out_sems = (so0, so1)

    def in_copy(r):
        return pltpu.make_async_copy(
            scores_hbm.at[wid * ROWS_PER_W + r], rbufs[r % 2], in_sems[r % 2]
        )

    def out_copy(r):
        return pltpu.make_async_copy(
            rbufs[r % 2], out_hbm.at[wid * ROWS_PER_W + r], out_sems[r % 2]
        )

    in_copy(0).start()
    for r in range(ROWS_PER_W):
        if r + 1 < ROWS_PER_W:
            if r >= 1:
                out_copy(r - 1).wait()
            in_copy(r + 1).start()
        in_copy(r).wait()
        process_row(rbufs[r % 2])
        out_copy(r).start()
    out_copy(ROWS_PER_W - 2).wait()
    out_copy(ROWS_PER_W - 1).wait()


@functools.partial(jax.jit, static_argnames=())
def kernel(scores, k):
    assert scores.shape == (NROWS, NCOLS) and scores.dtype == jnp.float32
    delta = jnp.asarray(k, jnp.float32) - jnp.float32(K_SEL)
    delta_arr = jnp.full((L,), delta, dtype=jnp.float32)

    mesh = plsc.VectorSubcoreMesh(core_axis_name="c", subcore_axis_name="s")
    fn = functools.partial(
        pl.kernel,
        mesh=mesh,
        compiler_params=pltpu.CompilerParams(needs_layout_passes=False),
        out_type=jax.ShapeDtypeStruct((NROWS, NCOLS), jnp.float32),
        scratch_types=[
            pltpu.VMEM((NCOLS,), jnp.float32),
            pltpu.VMEM((NCOLS,), jnp.float32),
            pltpu.VMEM((NCOLS,), jnp.int32),
            pltpu.VMEM((NB_A,), jnp.int32),
            pltpu.VMEM((NB_A,), jnp.int32),
            pltpu.VMEM((NB_A,), jnp.int32),
            pltpu.VMEM((NB_A,), jnp.int32),
            pltpu.VMEM((NB_S,), jnp.int32),
            pltpu.VMEM((L,), jnp.float32),
            pltpu.SemaphoreType.DMA,
            pltpu.SemaphoreType.DMA,
            pltpu.SemaphoreType.DMA,
            pltpu.SemaphoreType.DMA,
        ],
    )(_body)
    return fn(scores, delta_arr)


# speedup vs baseline: 1.0027x; 1.0027x over previous
import functools

import jax
import jax.numpy as jnp
from jax import lax
from jax.experimental import pallas as pl
from jax.experimental.pallas import tpu as pltpu
from jax.experimental.pallas import tpu_sc as plsc

NROWS = 128
NCOLS = 32768
K_SEL = 200
L = 16
NC = 2
NS = 16
NW = NC * NS
ROWS_PER_W = NROWS // NW
NV = NCOLS // L
UNROLL = 16
NOUTER = NV // UNROLL

NB_A = 512
NB_S = 128
LOW_MASK = (1 << 21) - 1


def _scan_hist(h_ref, nbuckets, rank, iota, fifteen):
    def body(j, carry):
        cum, found, bucket, cb = carry
        h = h_ref[pl.ds(j * L, L)]
        pc = plsc.cumsum(h)
        tot = pc + cum
        crossed = tot >= rank
        has = plsc.all_reduce_population_count(crossed) > 0
        ffs = plsc.all_reduce_ffs(crossed)
        is_new = jnp.logical_and(has, jnp.logical_not(found))
        bucket = jnp.where(is_new, j * L + ffs, bucket)
        cb = jnp.where(jnp.logical_and(is_new, iota == ffs), tot - h, cb)
        found = jnp.logical_or(found, has)
        cum = cum + jnp.take(pc, fifteen, mode="wrap")
        return cum, found, bucket, cb

    zs = jnp.zeros((L,), jnp.int32)
    init = (zs, zs > 0, zs, zs)
    _, _, bucket, cb = lax.fori_loop(0, nbuckets // L, body, init)
    return bucket, cb


def _body(
    scores_hbm,
    delta_hbm,
    out_hbm,
    rv0,
    rv1,
    buf,
    ha0,
    ha1,
    hs,
    delta_v,
    si0,
    si1,
    so0,
    so1,
):
    wid = lax.axis_index("s") * NC + lax.axis_index("c")
    pltpu.sync_copy(delta_hbm, delta_v)
    ones_i = jnp.ones((L,), jnp.int32)
    zeros_i = jnp.zeros((L,), jnp.int32)
    iota = lax.iota(jnp.int32, L)
    fifteen = jnp.full((L,), L - 1, jnp.int32)
    rank0 = jnp.full((L,), K_SEL, jnp.int32)
    big_i = jnp.full((L,), 0x7FFFFFFF, jnp.int32)

    def refine_level(shift, path, rank, nvb):
        for z in range(NB_S // L):
            hs[pl.ds(z * L, L)] = zeros_i

        def hist(j, _):
            low = buf[pl.ds(j * L, L)]
            m = lax.shift_right_logical(low, shift + 7) == path
            key = jnp.bitwise_and(lax.shift_right_logical(low, shift), NB_S - 1)
            plsc.addupdate_scatter(hs, [key], ones_i, mask=m)
            return 0

        lax.fori_loop(0, nvb, hist, 0)
        b, cb = _scan_hist(hs, NB_S, rank, iota, fifteen)
        return path * NB_S + b, rank - jnp.sum(cb)

    def process_row(row_v):
        has = (ha0, ha1)

        def zero_a(z, _):
            for h in has:
                h[pl.ds(z * L, L)] = zeros_i
                h[pl.ds(NB_A + z * L, L)] = zeros_i
            return 0

        lax.fori_loop(0, NB_A // L, zero_a, 0)

        sub_iota = jnp.bitwise_and(iota, 1) * NB_A

        def pass_a(i, _):
            bs = [
                sub_iota
                + lax.shift_right_logical(
                    lax.bitcast_convert_type(
                        row_v[pl.ds(i * (L * UNROLL) + u * L, L)], jnp.int32
                    ),
                    21,
                )
                for u in range(UNROLL)
            ]
            for u in range(UNROLL):
                plsc.addupdate_scatter(has[u % 2], [bs[u]], ones_i)
            return 0

        lax.fori_loop(0, NOUTER, pass_a, 0)

        def merge(j, _):
            sl = pl.ds(j * L, L)
            sh = pl.ds(NB_A + j * L, L)
            ha0[sl] = (ha0[sl] + ha0[sh]) + (ha1[sl] + ha1[sh])
            return 0

        lax.fori_loop(0, NB_A // L, merge, 0)
        pa, cb_a = _scan_hist(ha0, NB_A, rank0, iota, fifteen)
        rank = rank0 - jnp.sum(cb_a)

        def pass_b(i, off):
            bits_l = [
                lax.bitcast_convert_type(
                    row_v[pl.ds(i * (L * UNROLL) + u * L, L)], jnp.int32
                )
                for u in range(UNROLL)
            ]
            match_l = [lax.shift_right_logical(b, 21) == pa for b in bits_l]
            adv_l = [
                jnp.where(plsc.all_reduce_population_count(m) > 0, L, 0)
                for m in match_l
            ]
            offs = [off]
            for u in range(1, UNROLL):
                offs.append(offs[-1] + adv_l[u - 1])
            for u in range(UNROLL):
                plsc.store_scatter(buf, [offs[u] + iota], bits_l[u])
            return offs[-1] + adv_l[-1]

        off = lax.fori_loop(0, NOUTER, pass_b, zeros_i)
        nvb = jnp.max(off) // L

        path = pa
        path, rank = refine_level(14, path, rank, nvb)
        path, rank = refine_level(7, path, rank, nvb)
        path, _ = refine_level(0, path, rank, nvb)

        bits200 = path
        x200 = lax.bitcast_convert_type(bits200, jnp.float32)
        y200 = 1.0 - (2.0 * x200) / 0.05
        kth = (jnp.where(y200 > 0.0, y200, jnp.exp(y200) - 1.0) + 1.0) / 2.0
        kth = kth + delta_v[...]
        kth_s = jnp.max(kth)

        @pl.when(kth_s >= 0.5)
        def _():
            def out_fast(i, _):
                for u in range(UNROLL):
                    sl = pl.ds(i * (L * UNROLL) + u * L, L)
                    y = 1.0 - (2.0 * row_v[sl]) / 0.05
                    f = (y + 1.0) / 2.0
                    row_v[sl] = jnp.where(f >= kth, f, 0.0)
                return 0

            lax.fori_loop(0, NOUTER, out_fast, 0)

        @pl.when(kth_s < 0.5)
        def _():
            def out_full(i, _):
                for u in range(UNROLL):
                    sl = pl.ds(i * (L * UNROLL) + u * L, L)
                    y = 1.0 - (2.0 * row_v[sl]) / 0.05
                    c = jnp.where(y > 0.0, y, jnp.exp(y) - 1.0)
                    f = (c + 1.0) / 2.0
                    row_v[sl] = jnp.where(f >= kth, f, 0.0)
                return 0

            lax.fori_loop(0, NOUTER, out_full, 0)

    rbufs = (rv0, rv1)
    in_sems = (si0, si1)
    out_sems = (so0, so1)

    def in_copy(r):
        return pltpu.make_async_copy(
            scores_hbm.at[wid * ROWS_PER_W + r], rbufs[r % 2], in_sems[r % 2]
        )

    def out_copy(r):
        return pltpu.make_async_copy(
            rbufs[r % 2], out_hbm.at[wid * ROWS_PER_W + r], out_sems[r % 2]
        )

    in_copy(0).start()
    for r in range(ROWS_PER_W):
        if r + 1 < ROWS_PER_W:
            if r >= 1:
                out_copy(r - 1).wait()
            in_copy(r + 1).start()
        in_copy(r).wait()
        process_row(rbufs[r % 2])
        out_copy(r).start()
    out_copy(ROWS_PER_W - 2).wait()
    out_copy(ROWS_PER_W - 1).wait()


@functools.partial(jax.jit, static_argnames=())
def kernel(scores, k):
    assert scores.shape == (NROWS, NCOLS) and scores.dtype == jnp.float32
    delta = jnp.asarray(k, jnp.float32) - jnp.float32(K_SEL)
    delta_arr = jnp.full((L,), delta, dtype=jnp.float32)

    mesh = plsc.VectorSubcoreMesh(core_axis_name="c", subcore_axis_name="s")
    fn = functools.partial(
        pl.kernel,
        mesh=mesh,
        compiler_params=pltpu.CompilerParams(needs_layout_passes=False),
        out_type=jax.ShapeDtypeStruct((NROWS, NCOLS), jnp.float32),
        scratch_types=[
            pltpu.VMEM((NCOLS,), jnp.float32),
            pltpu.VMEM((NCOLS,), jnp.float32),
            pltpu.VMEM((NCOLS,), jnp.int32),
            pltpu.VMEM((2 * NB_A,), jnp.int32),
            pltpu.VMEM((2 * NB_A,), jnp.int32),
            pltpu.VMEM((NB_S,), jnp.int32),
            pltpu.VMEM((L,), jnp.float32),
            pltpu.SemaphoreType.DMA,
            pltpu.SemaphoreType.DMA,
            pltpu.SemaphoreType.DMA,
            pltpu.SemaphoreType.DMA,
        ],
    )(_body)
    return fn(scores, delta_arr)


# speedup vs baseline: 1.0192x; 1.0165x over previous
import functools

import jax
import jax.numpy as jnp
from jax import lax
from jax.experimental import pallas as pl
from jax.experimental.pallas import tpu as pltpu
from jax.experimental.pallas import tpu_sc as plsc

NROWS = 128
NCOLS = 32768
K_SEL = 200
L = 16
NC = 2
NS = 16
NW = NC * NS
ROWS_PER_W = NROWS // NW
NV = NCOLS // L
UNROLL = 16
NOUTER = NV // UNROLL

NB_A = 512
NB_S = 128


def _scan_hist(h_ref, nbuckets, rank, iota, fifteen):
    def body(j, carry):
        cum, found, bucket, cb = carry
        h = h_ref[pl.ds(j * L, L)]
        pc = plsc.cumsum(h)
        tot = pc + cum
        crossed = tot >= rank
        has = plsc.all_reduce_population_count(crossed) > 0
        ffs = plsc.all_reduce_ffs(crossed)
        is_new = jnp.logical_and(has, jnp.logical_not(found))
        bucket = jnp.where(is_new, j * L + ffs, bucket)
        cb = jnp.where(jnp.logical_and(is_new, iota == ffs), tot - h, cb)
        found = jnp.logical_or(found, has)
        cum = cum + jnp.take(pc, fifteen, mode="wrap")
        return cum, found, bucket, cb

    zs = jnp.zeros((L,), jnp.int32)
    init = (zs, zs > 0, zs, zs)
    _, _, bucket, cb = lax.fori_loop(0, nbuckets // L, body, init)
    return bucket, cb


def _body(
    scores_hbm,
    delta_hbm,
    out_hbm,
    rv0,
    rv1,
    buf,
    ha0,
    ha1,
    ha2,
    ha3,
    hs,
    delta_v,
    si0,
    si1,
    so0,
    so1,
):
    wid = lax.axis_index("s") * NC + lax.axis_index("c")
    pltpu.sync_copy(delta_hbm, delta_v)
    ones_i = jnp.ones((L,), jnp.int32)
    zeros_i = jnp.zeros((L,), jnp.int32)
    iota = lax.iota(jnp.int32, L)
    fifteen = jnp.full((L,), L - 1, jnp.int32)
    rank0 = jnp.full((L,), K_SEL, jnp.int32)

    def refine_level(shift, path, rank, nvb):
        for z in range(NB_S // L):
            hs[pl.ds(z * L, L)] = zeros_i

        def hist(j, _):
            low = buf[pl.ds(j * L, L)]
            m = lax.shift_right_logical(low, shift + 7) == path
            key = jnp.bitwise_and(lax.shift_right_logical(low, shift), NB_S - 1)
            plsc.addupdate_scatter(hs, [key], ones_i, mask=m)
            return 0

        lax.fori_loop(0, nvb, hist, 0)
        b, cb = _scan_hist(hs, NB_S, rank, iota, fifteen)
        return path * NB_S + b, rank - jnp.sum(cb)

    def process_row(row_v):
        has = (ha0, ha1, ha2, ha3)

        def zero_a(z, _):
            for h in has:
                h[pl.ds(z * L, L)] = zeros_i
            return 0

        lax.fori_loop(0, NB_A // L, zero_a, 0)

        def pass_a(i, _):
            bs = [
                lax.shift_right_logical(
                    lax.bitcast_convert_type(
                        row_v[pl.ds(i * (L * UNROLL) + u * L, L)], jnp.int32
                    ),
                    21,
                )
                for u in range(UNROLL)
            ]
            for u in range(UNROLL):
                plsc.addupdate_scatter(has[u % 4], [bs[u]], ones_i)
            return 0

        lax.fori_loop(0, NOUTER, pass_a, 0)

        def merge(j, _):
            sl = pl.ds(j * L, L)
            ha0[sl] = (ha0[sl] + ha1[sl]) + (ha2[sl] + ha3[sl])
            return 0

        lax.fori_loop(0, NB_A // L, merge, 0)
        pa, cb_a = _scan_hist(ha0, NB_A, rank0, iota, fifteen)
        rank = rank0 - jnp.sum(cb_a)

        def pass_b(i, off):
            bits_l = [
                lax.bitcast_convert_type(
                    row_v[pl.ds(i * (L * UNROLL) + u * L, L)], jnp.int32
                )
                for u in range(UNROLL)
            ]
            match_l = [lax.shift_right_logical(b, 21) == pa for b in bits_l]
            adv_l = [
                jnp.where(plsc.all_reduce_population_count(m) > 0, L, 0)
                for m in match_l
            ]
            offs = [off]
            for u in range(1, UNROLL):
                offs.append(offs[-1] + adv_l[u - 1])
            for u in range(UNROLL):
                plsc.store_scatter(buf, [offs[u] + iota], bits_l[u])
            return offs[-1] + adv_l[-1]

        off = lax.fori_loop(0, NOUTER, pass_b, zeros_i)
        nvb = jnp.max(off) // L

        path = pa
        path, rank = refine_level(14, path, rank, nvb)
        path, rank = refine_level(7, path, rank, nvb)
        path, _ = refine_level(0, path, rank, nvb)

        bits200 = path
        x200 = lax.bitcast_convert_type(bits200, jnp.float32)
        y200 = 1.0 - (2.0 * x200) / 0.05
        kth = (jnp.where(y200 > 0.0, y200, jnp.exp(y200) - 1.0) + 1.0) / 2.0
        kth = kth + delta_v[...]
        kth_s = jnp.max(kth)

        @pl.when(kth_s >= 0.5)
        def _():
            def out_fast(i, _):
                for u in range(UNROLL):
                    sl = pl.ds(i * (L * UNROLL) + u * L, L)
                    y = 1.0 - (2.0 * row_v[sl]) / 0.05
                    f = (y + 1.0) / 2.0
                    row_v[sl] = jnp.where(f >= kth, f, 0.0)
                return 0

            lax.fori_loop(0, NOUTER, out_fast, 0)

        @pl.when(kth_s < 0.5)
        def _():
            def out_full(i, _):
                for u in range(UNROLL):
                    sl = pl.ds(i * (L * UNROLL) + u * L, L)
                    y = 1.0 - (2.0 * row_v[sl]) / 0.05
                    c = jnp.where(y > 0.0, y, jnp.exp(y) - 1.0)
                    f = (c + 1.0) / 2.0
                    row_v[sl] = jnp.where(f >= kth, f, 0.0)
                return 0

            lax.fori_loop(0, NOUTER, out_full, 0)

    rbufs = (rv0, rv1)
    in_sems = (si0, si1)
    out_sems = (so0, so1)

    def in_copy(r):
        return pltpu.make_async_copy(
            scores_hbm.at[wid * ROWS_PER_W + r], rbufs[r % 2], in_sems[r % 2]
        )

    def out_copy(r):
        return pltpu.make_async_copy(
            rbufs[r % 2], out_hbm.at[wid * ROWS_PER_W + r], out_sems[r % 2]
        )

    in_copy(0).start()
    for r in range(ROWS_PER_W):
        if r + 1 < ROWS_PER_W:
            if r >= 1:
                out_copy(r - 1).wait()
            in_copy(r + 1).start()
        in_copy(r).wait()
        process_row(rbufs[r % 2])
        out_copy(r).start()
    out_copy(ROWS_PER_W - 2).wait()
    out_copy(ROWS_PER_W - 1).wait()


@functools.partial(jax.jit, static_argnames=())
def kernel(scores, k):
    assert scores.shape == (NROWS, NCOLS) and scores.dtype == jnp.float32
    delta = jnp.asarray(k, jnp.float32) - jnp.float32(K_SEL)
    delta_arr = jnp.full((L,), delta, dtype=jnp.float32)

    mesh = plsc.VectorSubcoreMesh(core_axis_name="c", subcore_axis_name="s")
    fn = functools.partial(
        pl.kernel,
        mesh=mesh,
        compiler_params=pltpu.CompilerParams(needs_layout_passes=False),
        out_type=jax.ShapeDtypeStruct((NROWS, NCOLS), jnp.float32),
        scratch_types=[
            pltpu.VMEM((NCOLS,), jnp.float32),
            pltpu.VMEM((NCOLS,), jnp.float32),
            pltpu.VMEM((NCOLS,), jnp.int32),
            pltpu.VMEM((NB_A,), jnp.int32),
            pltpu.VMEM((NB_A,), jnp.int32),
            pltpu.VMEM((NB_A,), jnp.int32),
            pltpu.VMEM((NB_A,), jnp.int32),
            pltpu.VMEM((NB_S,), jnp.int32),
            pltpu.VMEM((L,), jnp.float32),
            pltpu.SemaphoreType.DMA,
            pltpu.SemaphoreType.DMA,
            pltpu.SemaphoreType.DMA,
            pltpu.SemaphoreType.DMA,
        ],
    )(_body)
    return fn(scores, delta_arr)
